# baseline (speedup 1.0000x reference)
import functools
import jax
import jax.numpy as jnp
from jax import lax
from jax.experimental import pallas as pl
from jax.experimental.pallas import tpu as pltpu
from jax.experimental.pallas import tpu_sc as plsc

_BS, _N, _C = 8, 4096, 128
_HID, _OUT = 64, 16
_U = 16
_R = _BS * _N
_NC, _NS = 2, 16
_NW = _NC * _NS
_RPW = _R // _NW
_CHUNK = 128
_NCHUNK = _RPW // _CHUNK
_ZROWS = 9
_ACC_ROWS = _NS * _ZROWS
_DUMP = _BS * _U


def _sc_segsum(vals2d, orb_flat, mask_flat):
    mesh = plsc.VectorSubcoreMesh(core_axis_name="c", subcore_axis_name="s")

    @functools.partial(
        pl.kernel,
        mesh=mesh,
        out_type=jax.ShapeDtypeStruct((_NC, _BS * _U, _C), jnp.float32),
        scratch_types=[
            pltpu.VMEM((_CHUNK, _C), jnp.float32),
            pltpu.VMEM((_RPW,), jnp.int32),
            pltpu.VMEM((_RPW,), jnp.int32),
            pltpu.VMEM((_CHUNK,), jnp.int32),
            pltpu.VMEM_SHARED((_ACC_ROWS, _C), jnp.float32),
        ],
    )
    def k(vals_hbm, orb_hbm, mask_hbm, out_hbm, row_v, orb_v, mask_v,
          idx_v, acc_s):
        c = lax.axis_index("c")
        s = lax.axis_index("s")
        wid = s * _NC + c
        base = wid * _RPW
        b16 = (base // _N) * _U

        z16 = jnp.zeros((16,), jnp.float32)
        for j in range(_ZROWS):
            for kk in range(_C // 16):
                row_v[j, pl.ds(kk * 16, 16)] = z16
        pltpu.sync_copy(row_v.at[pl.ds(0, _ZROWS)],
                        acc_s.at[pl.ds(s * _ZROWS, _ZROWS)])
        plsc.subcore_barrier()

        pltpu.sync_copy(orb_hbm.at[pl.ds(base, _RPW)], orb_v)
        pltpu.sync_copy(mask_hbm.at[pl.ds(base, _RPW)], mask_v)
        for g in range(_NCHUNK):
            pltpu.sync_copy(vals_hbm.at[pl.ds(base + g * _CHUNK, _CHUNK)],
                            row_v)
            for kk in range(_CHUNK // 16):
                off = g * _CHUNK + kk * 16
                m = mask_v[pl.ds(off, 16)]
                o = orb_v[pl.ds(off, 16)]
                idx_v[pl.ds(kk * 16, 16)] = jnp.where(m > 0, o + b16, _DUMP)
            pltpu.sync_copy(row_v, acc_s.at[idx_v], add=True)
        plsc.subcore_barrier()

        pltpu.sync_copy(acc_s.at[pl.ds(s * 8, 8)], row_v.at[pl.ds(0, 8)])
        pltpu.sync_copy(row_v.at[pl.ds(0, 8)],
                        out_hbm.at[c, pl.ds(s * 8, 8)])

    return k(vals2d, orb_flat, mask_flat)


def _mlp_body(p_ref, W1_ref, b1_ref, W2_ref, b2_ref, W3_ref, b3_ref,
              out_ref):
    a = p_ref[0] + p_ref[1]
    rowsum = jnp.sum(a, axis=1, keepdims=True)
    empty = rowsum == 0.0
    h = jax.nn.relu(jnp.dot(a, W1_ref[...],
                            preferred_element_type=jnp.float32) + b1_ref[...])
    h = jax.nn.relu(jnp.dot(h, W2_ref[...],
                            preferred_element_type=jnp.float32) + b2_ref[...])
    t = jnp.dot(h, W3_ref[...], preferred_element_type=jnp.float32) + b3_ref[...]
    t = jnp.where(empty, 0.0, t)
    col = lax.broadcasted_iota(jnp.int32, (_BS, _BS * _U), 1)
    row = lax.broadcasted_iota(jnp.int32, (_BS, _BS * _U), 0)
    sel = (col // _U == row).astype(jnp.float32)
    out_ref[...] = jnp.dot(sel, t, preferred_element_type=jnp.float32)


def kernel(coords, vals, mask, W1, b1, W2, b2, W3, b3):
    vals2d = vals.reshape(_R, _C)
    orb_flat = coords[:, :, 1, 1].astype(jnp.int32).reshape(_R)
    mask_flat = mask.astype(jnp.int32).reshape(_R)

    partials = _sc_segsum(vals2d, orb_flat, mask_flat)

    out = pl.pallas_call(
        _mlp_body,
        out_shape=jax.ShapeDtypeStruct((_BS, _OUT), jnp.float32),
    )(partials, W1, b1.reshape(1, _HID), W2, b2.reshape(1, _HID),
      W3, b3.reshape(1, _OUT))
    return out


# speedup vs baseline: 1.1049x; 1.1049x over previous
import functools
import jax
import jax.numpy as jnp
from jax import lax
from jax.experimental import pallas as pl
from jax.experimental.pallas import tpu as pltpu
from jax.experimental.pallas import tpu_sc as plsc

_BS, _N, _C = 8, 4096, 128
_HID, _OUT = 64, 16
_U = 16
_R = _BS * _N
_NC, _NS = 2, 16
_NW = _NC * _NS
_RPW = _R // _NW
_CHUNK = 128
_NCHUNK = _RPW // _CHUNK
_ZROWS = 9
_ACC_ROWS = _NS * _ZROWS
_DUMP = _BS * _U


def _sc_segsum(vals2d, orb_flat, mask_flat):
    mesh = plsc.VectorSubcoreMesh(core_axis_name="c", subcore_axis_name="s")

    @functools.partial(
        pl.kernel,
        mesh=mesh,
        out_type=jax.ShapeDtypeStruct((_NC, _BS * _U, _C), jnp.float32),
        scratch_types=[
            pltpu.VMEM((_CHUNK, _C), jnp.float32),
            pltpu.VMEM((_CHUNK, _C), jnp.float32),
            pltpu.VMEM((_RPW,), jnp.int32),
            pltpu.VMEM((_RPW,), jnp.int32),
            pltpu.VMEM((_CHUNK,), jnp.int32),
            pltpu.VMEM((_CHUNK,), jnp.int32),
            pltpu.VMEM_SHARED((_ACC_ROWS, _C), jnp.float32),
            pltpu.SemaphoreType.DMA,
            pltpu.SemaphoreType.DMA,
            pltpu.SemaphoreType.DMA,
            pltpu.SemaphoreType.DMA,
        ],
    )
    def k(vals_hbm, orb_hbm, mask_hbm, out_hbm, row_a, row_b, orb_v,
          mask_v, idx_a, idx_b, acc_s, lsem_a, lsem_b, ssem_a, ssem_b):
        c = lax.axis_index("c")
        s = lax.axis_index("s")
        wid = s * _NC + c
        base = wid * _RPW
        b16 = (base // _N) * _U

        rows = (row_a, row_b)
        idxs = (idx_a, idx_b)
        lsems = (lsem_a, lsem_b)
        ssems = (ssem_a, ssem_b)

        z16 = jnp.zeros((16,), jnp.float32)
        for j in range(_ZROWS):
            for kk in range(_C // 16):
                row_a[j, pl.ds(kk * 16, 16)] = z16
        pltpu.sync_copy(row_a.at[pl.ds(0, _ZROWS)],
                        acc_s.at[pl.ds(s * _ZROWS, _ZROWS)])

        pltpu.sync_copy(orb_hbm.at[pl.ds(base, _RPW)], orb_v)
        pltpu.sync_copy(mask_hbm.at[pl.ds(base, _RPW)], mask_v)
        plsc.subcore_barrier()

        loads = [None, None]
        scats = [None, None]
        for g in range(_NCHUNK + 1):
            if g < _NCHUNK:
                buf = g % 2
                if scats[buf] is not None:
                    scats[buf].wait()
                loads[buf] = pltpu.async_copy(
                    vals_hbm.at[pl.ds(base + g * _CHUNK, _CHUNK)],
                    rows[buf], lsems[buf])
            if g >= 1:
                gg = g - 1
                buf = gg % 2
                for kk in range(_CHUNK // 16):
                    off = gg * _CHUNK + kk * 16
                    m = mask_v[pl.ds(off, 16)]
                    o = orb_v[pl.ds(off, 16)]
                    idxs[buf][pl.ds(kk * 16, 16)] = jnp.where(
                        m > 0, o + b16, _DUMP)
                loads[buf].wait()
                scats[buf] = pltpu.async_copy(
                    rows[buf], acc_s.at[idxs[buf]], ssems[buf], add=True)
        scats[0].wait()
        scats[1].wait()
        plsc.subcore_barrier()

        pltpu.sync_copy(acc_s.at[pl.ds(s * 8, 8)], row_a.at[pl.ds(0, 8)])
        pltpu.sync_copy(row_a.at[pl.ds(0, 8)],
                        out_hbm.at[c, pl.ds(s * 8, 8)])

    return k(vals2d, orb_flat, mask_flat)


def _mlp_body(p_ref, W1_ref, b1_ref, W2_ref, b2_ref, W3_ref, b3_ref,
              out_ref):
    a = p_ref[0] + p_ref[1]
    rowsum = jnp.sum(a, axis=1, keepdims=True)
    empty = rowsum == 0.0
    h = jax.nn.relu(jnp.dot(a, W1_ref[...],
                            preferred_element_type=jnp.float32) + b1_ref[...])
    h = jax.nn.relu(jnp.dot(h, W2_ref[...],
                            preferred_element_type=jnp.float32) + b2_ref[...])
    t = jnp.dot(h, W3_ref[...], preferred_element_type=jnp.float32) + b3_ref[...]
    t = jnp.where(empty, 0.0, t)
    col = lax.broadcasted_iota(jnp.int32, (_BS, _BS * _U), 1)
    row = lax.broadcasted_iota(jnp.int32, (_BS, _BS * _U), 0)
    sel = (col // _U == row).astype(jnp.float32)
    out_ref[...] = jnp.dot(sel, t, preferred_element_type=jnp.float32)


def kernel(coords, vals, mask, W1, b1, W2, b2, W3, b3):
    vals2d = vals.reshape(_R, _C)
    orb_flat = coords[:, :, 1, 1].astype(jnp.int32).reshape(_R)
    mask_flat = mask.astype(jnp.int32).reshape(_R)

    partials = _sc_segsum(vals2d, orb_flat, mask_flat)

    out = pl.pallas_call(
        _mlp_body,
        out_shape=jax.ShapeDtypeStruct((_BS, _OUT), jnp.float32),
    )(partials, W1, b1.reshape(1, _HID), W2, b2.reshape(1, _HID),
      W3, b3.reshape(1, _OUT))
    return out


# speedup vs baseline: 2.4365x; 2.2053x over previous
import jax
import jax.numpy as jnp
from jax import lax
from jax.experimental import pallas as pl
from jax.experimental.pallas import tpu as pltpu

_BS, _N, _C = 8, 4096, 128
_HID, _OUT = 64, 16
_U = 16


def _body(orb_ref, maskf_ref, vals_ref, W1_ref, b1_ref, W2_ref, b2_ref,
          W3_ref, b3_ref, out_ref):
    orb = orb_ref[0]
    maskf = maskf_ref[0]
    orb_b = jnp.broadcast_to(orb, (_U, _N))
    row_u = lax.broadcasted_iota(jnp.int32, (_U, _N), 0)
    ohT = jnp.where(orb_b == row_u, jnp.broadcast_to(maskf, (_U, _N)), 0.0)
    agg = lax.dot_general(ohT, vals_ref[0],
                          (((1,), (0,)), ((), ())),
                          preferred_element_type=jnp.float32)
    rowsum = jnp.sum(agg, axis=1, keepdims=True)
    empty = rowsum == 0.0
    h = jax.nn.relu(jnp.dot(agg, W1_ref[...],
                            preferred_element_type=jnp.float32) + b1_ref[...])
    h = jax.nn.relu(jnp.dot(h, W2_ref[...],
                            preferred_element_type=jnp.float32) + b2_ref[...])
    t = jnp.dot(h, W3_ref[...], preferred_element_type=jnp.float32) + b3_ref[...]
    t = jnp.where(empty, 0.0, t)
    out_ref[0] = jnp.sum(t, axis=0, keepdims=True)


def kernel(coords, vals, mask, W1, b1, W2, b2, W3, b3):
    orb_ids = coords[:, :, 1, 1].astype(jnp.int32).reshape(_BS, 1, _N)
    maskf = mask.astype(jnp.float32).reshape(_BS, 1, _N)
    b1r = b1.reshape(1, _HID)
    b2r = b2.reshape(1, _HID)
    b3r = b3.reshape(1, _OUT)

    out = pl.pallas_call(
        _body,
        grid=(_BS,),
        in_specs=[
            pl.BlockSpec((1, 1, _N), lambda b: (b, 0, 0)),
            pl.BlockSpec((1, 1, _N), lambda b: (b, 0, 0)),
            pl.BlockSpec((1, _N, _C), lambda b: (b, 0, 0)),
            pl.BlockSpec((_C, _HID), lambda b: (0, 0)),
            pl.BlockSpec((1, _HID), lambda b: (0, 0)),
            pl.BlockSpec((_HID, _HID), lambda b: (0, 0)),
            pl.BlockSpec((1, _HID), lambda b: (0, 0)),
            pl.BlockSpec((_HID, _OUT), lambda b: (0, 0)),
            pl.BlockSpec((1, _OUT), lambda b: (0, 0)),
        ],
        out_specs=pl.BlockSpec((1, 1, _OUT), lambda b: (b, 0, 0)),
        out_shape=jax.ShapeDtypeStruct((_BS, 1, _OUT), jnp.float32),
    )(orb_ids, maskf, vals, W1, b1r, W2, b2r, W3, b3r)
    return out.reshape(_BS, _OUT)


# speedup vs baseline: 2.7060x; 1.1106x over previous
import jax
import jax.numpy as jnp
from jax import lax
from jax.experimental import pallas as pl
from jax.experimental.pallas import tpu as pltpu

_BS, _N, _C = 8, 4096, 128
_HID, _OUT = 64, 16
_U = 16


def _body(morb_ref, vals_ref, W1_ref, b1_ref, W2_ref, b2_ref,
          W3_ref, b3_ref, out_ref):
    morb = morb_ref[0]
    morb_b = jnp.broadcast_to(morb, (_U, _N))
    row_u = lax.broadcasted_iota(jnp.int32, (_U, _N), 0)
    ohT = jnp.where(morb_b == row_u, 1.0, 0.0)
    agg = lax.dot_general(ohT, vals_ref[0],
                          (((1,), (0,)), ((), ())),
                          preferred_element_type=jnp.float32)
    rowsum = jnp.sum(agg, axis=1, keepdims=True)
    empty = rowsum == 0.0
    h = jax.nn.relu(jnp.dot(agg, W1_ref[...],
                            preferred_element_type=jnp.float32) + b1_ref[...])
    h = jax.nn.relu(jnp.dot(h, W2_ref[...],
                            preferred_element_type=jnp.float32) + b2_ref[...])
    t = jnp.dot(h, W3_ref[...], preferred_element_type=jnp.float32) + b3_ref[...]
    t = jnp.where(empty, 0.0, t)
    out_ref[0] = jnp.sum(t, axis=0, keepdims=True)


def kernel(coords, vals, mask, W1, b1, W2, b2, W3, b3):
    morb = jnp.where(mask[:, None, :], coords[:, :, 1, 1].reshape(_BS, 1, _N),
                     jnp.int32(_U)).astype(jnp.int32)
    b1r = b1.reshape(1, _HID)
    b2r = b2.reshape(1, _HID)
    b3r = b3.reshape(1, _OUT)

    out = pl.pallas_call(
        _body,
        grid=(_BS,),
        in_specs=[
            pl.BlockSpec((1, 1, _N), lambda b: (b, 0, 0)),
            pl.BlockSpec((1, _N, _C), lambda b: (b, 0, 0)),
            pl.BlockSpec((_C, _HID), lambda b: (0, 0)),
            pl.BlockSpec((1, _HID), lambda b: (0, 0)),
            pl.BlockSpec((_HID, _HID), lambda b: (0, 0)),
            pl.BlockSpec((1, _HID), lambda b: (0, 0)),
            pl.BlockSpec((_HID, _OUT), lambda b: (0, 0)),
            pl.BlockSpec((1, _OUT), lambda b: (0, 0)),
        ],
        out_specs=pl.BlockSpec((1, 1, _OUT), lambda b: (b, 0, 0)),
        out_shape=jax.ShapeDtypeStruct((_BS, 1, _OUT), jnp.float32),
    )(morb, vals, W1, b1r, W2, b2r, W3, b3r)
    return out.reshape(_BS, _OUT)


# speedup vs baseline: 2.7208x; 1.0055x over previous
import jax
import jax.numpy as jnp
from jax import lax
from jax.experimental import pallas as pl
from jax.experimental.pallas import tpu as pltpu

_BS, _N, _C = 8, 4096, 128
_HID, _OUT = 64, 16
_U = 16


def _body(morb_ref, vals_ref, W1_ref, b1_ref, W2_ref, b2_ref,
          W3_ref, b3_ref, out_ref):
    b = pl.program_id(0)
    morb = morb_ref[pl.ds(b, 1), :]
    morb_b = jnp.broadcast_to(morb, (_U, _N))
    row_u = lax.broadcasted_iota(jnp.int32, (_U, _N), 0)
    ohT = jnp.where(morb_b == row_u, 1.0, 0.0)
    agg = lax.dot_general(ohT, vals_ref[0],
                          (((1,), (0,)), ((), ())),
                          preferred_element_type=jnp.float32)
    rowsum = jnp.sum(agg, axis=1, keepdims=True)
    empty = rowsum == 0.0
    h = jax.nn.relu(jnp.dot(agg, W1_ref[...],
                            preferred_element_type=jnp.float32) + b1_ref[...])
    h = jax.nn.relu(jnp.dot(h, W2_ref[...],
                            preferred_element_type=jnp.float32) + b2_ref[...])
    t = jnp.dot(h, W3_ref[...], preferred_element_type=jnp.float32) + b3_ref[...]
    t = jnp.where(empty, 0.0, t)
    out_ref[0] = jnp.sum(t, axis=0, keepdims=True)


def kernel(coords, vals, mask, W1, b1, W2, b2, W3, b3):
    morb = jnp.where(mask, coords[:, :, 1, 1], jnp.int32(_U)).astype(jnp.int32)
    b1r = b1.reshape(1, _HID)
    b2r = b2.reshape(1, _HID)
    b3r = b3.reshape(1, _OUT)

    out = pl.pallas_call(
        _body,
        grid=(_BS,),
        in_specs=[
            pl.BlockSpec((_BS, _N), lambda b: (0, 0)),
            pl.BlockSpec((1, _N, _C), lambda b: (b, 0, 0)),
            pl.BlockSpec((_C, _HID), lambda b: (0, 0)),
            pl.BlockSpec((1, _HID), lambda b: (0, 0)),
            pl.BlockSpec((_HID, _HID), lambda b: (0, 0)),
            pl.BlockSpec((1, _HID), lambda b: (0, 0)),
            pl.BlockSpec((_HID, _OUT), lambda b: (0, 0)),
            pl.BlockSpec((1, _OUT), lambda b: (0, 0)),
        ],
        out_specs=pl.BlockSpec((1, 1, _OUT), lambda b: (b, 0, 0)),
        out_shape=jax.ShapeDtypeStruct((_BS, 1, _OUT), jnp.float32),
    )(morb, vals, W1, b1r, W2, b2r, W3, b3r)
    return out.reshape(_BS, _OUT)


# speedup vs baseline: 3.0109x; 1.1066x over previous
import jax
import jax.numpy as jnp
from jax import lax
from jax.experimental import pallas as pl
from jax.experimental.pallas import tpu as pltpu

_BS, _N, _C = 8, 4096, 128
_HID, _OUT = 64, 16
_U = 16


def _body(morb_ref, vals_ref, W1_ref, b1_ref, W2_ref, b2_ref,
          W3_ref, b3_ref, out_ref):
    b = pl.program_id(0)
    morb = morb_ref[pl.ds(b, 1), :]
    morb_b = jnp.broadcast_to(morb, (_U, _N))
    row_u = lax.broadcasted_iota(jnp.int32, (_U, _N), 0)
    ohT = jnp.where(morb_b == row_u, 1.0, 0.0)
    agg = lax.dot_general(ohT, vals_ref[0],
                          (((1,), (0,)), ((), ())),
                          preferred_element_type=jnp.float32)
    rowsum = jnp.sum(agg, axis=1, keepdims=True)
    empty = rowsum == 0.0
    h = jax.nn.relu(lax.dot_general(agg, W1_ref[...], (((1,), (1,)), ((), ())),
                                    preferred_element_type=jnp.float32)
                    + b1_ref[...])
    h = jax.nn.relu(lax.dot_general(h, W2_ref[...], (((1,), (1,)), ((), ())),
                                    preferred_element_type=jnp.float32)
                    + b2_ref[...])
    t = lax.dot_general(h, W3_ref[...], (((1,), (1,)), ((), ())),
                        preferred_element_type=jnp.float32) + b3_ref[...]
    t = jnp.where(empty, 0.0, t)
    out_ref[0] = jnp.sum(t, axis=0, keepdims=True)


def kernel(coords, vals, mask, W1, b1, W2, b2, W3, b3):
    morb = jnp.where(mask, coords[:, :, 1, 1], jnp.int32(_U)).astype(jnp.int32)
    W1t, W2t, W3t = W1.T, W2.T, W3.T
    b1r = b1.reshape(1, _HID)
    b2r = b2.reshape(1, _HID)
    b3r = b3.reshape(1, _OUT)

    out = pl.pallas_call(
        _body,
        grid=(_BS,),
        in_specs=[
            pl.BlockSpec((_BS, _N), lambda b: (0, 0)),
            pl.BlockSpec((1, _N, _C), lambda b: (b, 0, 0)),
            pl.BlockSpec((_HID, _C), lambda b: (0, 0)),
            pl.BlockSpec((1, _HID), lambda b: (0, 0)),
            pl.BlockSpec((_HID, _HID), lambda b: (0, 0)),
            pl.BlockSpec((1, _HID), lambda b: (0, 0)),
            pl.BlockSpec((_OUT, _HID), lambda b: (0, 0)),
            pl.BlockSpec((1, _OUT), lambda b: (0, 0)),
        ],
        out_specs=pl.BlockSpec((1, 1, _OUT), lambda b: (b, 0, 0)),
        out_shape=jax.ShapeDtypeStruct((_BS, 1, _OUT), jnp.float32),
    )(morb, vals, W1t, b1r, W2t, b2r, W3t, b3r)
    return out.reshape(_BS, _OUT)
